# baseline (speedup 1.0000x reference)
import functools

import jax
import jax.numpy as jnp
from jax.experimental import pallas as pl
from jax.experimental.pallas import tpu as pltpu

_BLK = 256


def _gnar_block_kernel(coef_ref, a_ref, x_ref, y_ref, a8_ref, xa_ref,
                       s2_ref, *, n_lags: int):
    i = pl.program_id(0)
    n_blocks = pl.num_programs(0)
    Kn = a_ref.shape[0]
    Tn = x_ref.shape[1]
    blk = y_ref.shape[0]

    @pl.when(i == 0)
    def _prep():
        a8_ref[...] = a_ref[...].astype(jnp.float8_e4m3fn)
        col = jax.lax.broadcasted_iota(jnp.int32, (Kn, Tn), 1)
        xa_ref[...] = jnp.where(col == Tn - 1, 1.0,
                                x_ref[...]).astype(jnp.bfloat16)
        s2_ref[...] = jnp.zeros((Kn, Tn), dtype=jnp.float32)

    rows = pl.ds(i * blk, blk)
    a8_i = a8_ref[rows, :]
    xa_i = xa_ref[rows, :]

    def _pair(j, carry):
        cols = pl.ds(j * blk, blk)
        reach = jax.lax.dot_general(
            a8_i, a8_ref[:, cols], (((1,), (0,)), ((), ())),
            preferred_element_type=jnp.float32)
        a_ij = a_ref[rows, cols]
        m2 = jnp.maximum(jnp.minimum(reach, 1.0) - a_ij,
                         0.0).astype(jnp.bfloat16)
        xa_j = xa_ref[cols, :]
        s2_ref[rows, :] += jax.lax.dot_general(
            m2, xa_j, (((1,), (0,)), ((), ())),
            preferred_element_type=jnp.float32)

        @pl.when(j != i)
        def _transposed():
            s2_ref[cols, :] += jax.lax.dot_general(
                m2, xa_i, (((0,), (0,)), ((), ())),
                preferred_element_type=jnp.float32)

        return carry

    jax.lax.fori_loop(i, n_blocks, _pair, 0)

    S1 = jax.lax.dot_general(a_ref[rows, :].astype(jnp.bfloat16),
                             xa_ref[...], (((1,), (0,)), ((), ())),
                             preferred_element_type=jnp.float32)
    S2 = s2_ref[rows, :]

    c1 = S1[:, Tn - 1:Tn]
    own = (c1 > 0.0).astype(jnp.float32)
    c2 = S2[:, Tn - 1:Tn] - own
    inv1 = 1.0 / jnp.maximum(c1, 1.0)
    inv2 = 1.0 / jnp.maximum(c2, 1.0)
    own_i2 = own * inv2

    xb = x_ref[rows, :]
    P = n_lags
    y = jnp.zeros((blk, Tn - P), dtype=jnp.float32)
    for lag in range(1, P + 1):
        al = coef_ref[0, lag - 1]
        b0l = coef_ref[1, lag - 1]
        b1l = coef_ref[2, lag - 1]
        full = ((al - b1l * own_i2) * xb
                + (b0l * inv1) * S1
                + (b1l * inv2) * S2)
        lo, hi = P - lag, Tn - lag
        y = y + full[:, lo:hi]
    y_ref[...] = y


def kernel(X, A, alpha, beta0, beta1):
    Kn, Tn = X.shape
    P = alpha.shape[0]
    coef = jnp.stack([alpha, beta0, beta1]).astype(jnp.float32)

    blk = min(_BLK, Kn)
    grid = (Kn // blk,)
    body = functools.partial(_gnar_block_kernel, n_lags=P)

    return pl.pallas_call(
        body,
        grid=grid,
        in_specs=[
            pl.BlockSpec((3, P), lambda i: (0, 0)),
            pl.BlockSpec((Kn, Kn), lambda i: (0, 0)),
            pl.BlockSpec((Kn, Tn), lambda i: (0, 0)),
        ],
        out_specs=pl.BlockSpec((blk, Tn - P), lambda i: (i, 0)),
        out_shape=jax.ShapeDtypeStruct((Kn, Tn - P), jnp.float32),
        scratch_shapes=[
            pltpu.VMEM((Kn, Kn), jnp.float8_e4m3fn),
            pltpu.VMEM((Kn, Tn), jnp.bfloat16),
            pltpu.VMEM((Kn, Tn), jnp.float32),
        ],
    )(coef, A, X)


# speedup vs baseline: 1.9060x; 1.9060x over previous
import functools

import jax
import jax.numpy as jnp
from jax.experimental import pallas as pl
from jax.experimental.pallas import tpu as pltpu

_BLK = 256


def _gnar_kernel(coef_ref, a_ref, x_ref, y_ref, a8_ref, xa_ref, s2_ref,
                 *, n_lags: int, blk: int):
    Kn = a_ref.shape[0]
    Tn = x_ref.shape[1]
    n_blocks = Kn // blk
    P = n_lags

    a8_ref[...] = a_ref[...].astype(jnp.float8_e4m3fn)
    col = jax.lax.broadcasted_iota(jnp.int32, (Kn, Tn), 1)
    xa_ref[...] = jnp.where(col == Tn - 1, 1.0,
                            x_ref[...]).astype(jnp.bfloat16)
    s2_ref[...] = jnp.zeros((Kn, Tn), dtype=jnp.float32)

    for I in range(n_blocks):
        lo_r, hi_r = I * blk, (I + 1) * blk
        W = Kn - lo_r

        a8_i = a8_ref[lo_r:hi_r, :]
        xa_i = xa_ref[lo_r:hi_r, :]

        reach = jax.lax.dot_general(
            a8_i, a8_ref[:, lo_r:Kn], (((1,), (0,)), ((), ())),
            preferred_element_type=jnp.float32)
        m2 = jnp.maximum(
            jnp.minimum(reach, 1.0) - a_ref[lo_r:hi_r, lo_r:Kn],
            0.0).astype(jnp.bfloat16)

        xa_strip = xa_ref[lo_r:Kn, :]
        direct = jax.lax.dot_general(
            m2, xa_strip, (((1,), (0,)), ((), ())),
            preferred_element_type=jnp.float32)
        transposed = jax.lax.dot_general(
            m2, xa_i, (((0,), (0,)), ((), ())),
            preferred_element_type=jnp.float32)
        m2_ii = m2[:, :blk]
        comp = jax.lax.dot_general(
            m2_ii, xa_i, (((1,), (0,)), ((), ())),
            preferred_element_type=jnp.float32)

        s2_ref[lo_r:Kn, :] += transposed
        S2 = s2_ref[lo_r:hi_r, :] + direct - comp
        s2_ref[lo_r:hi_r, :] = S2

        S1 = jax.lax.dot_general(
            a_ref[lo_r:hi_r, :].astype(jnp.bfloat16), xa_ref[...],
            (((1,), (0,)), ((), ())),
            preferred_element_type=jnp.float32)

        c1 = S1[:, Tn - 1:Tn]
        own = (c1 > 0.0).astype(jnp.float32)
        c2 = S2[:, Tn - 1:Tn] - own
        inv1 = 1.0 / jnp.maximum(c1, 1.0)
        inv2 = 1.0 / jnp.maximum(c2, 1.0)
        own_i2 = own * inv2

        xb = x_ref[lo_r:hi_r, :]
        y = jnp.zeros((blk, Tn - P), dtype=jnp.float32)
        for lag in range(1, P + 1):
            al = coef_ref[0, lag - 1]
            b0l = coef_ref[1, lag - 1]
            b1l = coef_ref[2, lag - 1]
            full = ((al - b1l * own_i2) * xb
                    + (b0l * inv1) * S1
                    + (b1l * inv2) * S2)
            lo, hi = P - lag, Tn - lag
            y = y + full[:, lo:hi]
        y_ref[lo_r:hi_r, :] = y


def kernel(X, A, alpha, beta0, beta1):
    Kn, Tn = X.shape
    P = alpha.shape[0]
    coef = jnp.stack([alpha, beta0, beta1]).astype(jnp.float32)

    blk = min(_BLK, Kn)
    body = functools.partial(_gnar_kernel, n_lags=P, blk=blk)

    return pl.pallas_call(
        body,
        grid=(1,),
        in_specs=[
            pl.BlockSpec((3, P), lambda i: (0, 0)),
            pl.BlockSpec((Kn, Kn), lambda i: (0, 0)),
            pl.BlockSpec((Kn, Tn), lambda i: (0, 0)),
        ],
        out_specs=pl.BlockSpec((Kn, Tn - P), lambda i: (0, 0)),
        out_shape=jax.ShapeDtypeStruct((Kn, Tn - P), jnp.float32),
        scratch_shapes=[
            pltpu.VMEM((Kn, Kn), jnp.float8_e4m3fn),
            pltpu.VMEM((Kn, Tn), jnp.bfloat16),
            pltpu.VMEM((Kn, Tn), jnp.float32),
        ],
    )(coef, A, X)


# speedup vs baseline: 1.9317x; 1.0135x over previous
import functools

import jax
import jax.numpy as jnp
from jax.experimental import pallas as pl
from jax.experimental.pallas import tpu as pltpu

_BLK = 256


def _gnar_kernel(coef_ref, a_hbm, x_ref, y_ref, a_vmem, a8_ref, xa_ref,
                 s2_ref, sems, *, n_lags: int, blk: int):
    Kn = a_vmem.shape[0]
    Tn = x_ref.shape[1]
    n_blocks = Kn // blk
    P = n_lags

    copies = {}
    for I in range(n_blocks - 1, -1, -1):
        rows = pl.ds(I * blk, blk)
        copies[I] = pltpu.make_async_copy(
            a_hbm.at[rows, :], a_vmem.at[rows, :], sems.at[I])
        copies[I].start()

    col = jax.lax.broadcasted_iota(jnp.int32, (Kn, Tn), 1)
    xa_ref[...] = jnp.where(col == Tn - 1, 1.0,
                            x_ref[...]).astype(jnp.bfloat16)
    s2_ref[...] = jnp.zeros((Kn, Tn), dtype=jnp.float32)

    for I in range(n_blocks - 1, -1, -1):
        copies[I].wait()
        lo_r, hi_r = I * blk, (I + 1) * blk
        a8_ref[lo_r:hi_r, :] = a_vmem[lo_r:hi_r, :].astype(
            jnp.float8_e4m3fn)

        a8_i = a8_ref[lo_r:hi_r, :]
        xa_i = xa_ref[lo_r:hi_r, :]

        reach = jax.lax.dot_general(
            a8_i, a8_ref[lo_r:Kn, :], (((1,), (1,)), ((), ())),
            preferred_element_type=jnp.float32)
        m2 = jnp.maximum(
            jnp.minimum(reach, 1.0) - a_vmem[lo_r:hi_r, lo_r:Kn],
            0.0).astype(jnp.bfloat16)

        xa_strip = xa_ref[lo_r:Kn, :]
        direct = jax.lax.dot_general(
            m2, xa_strip, (((1,), (0,)), ((), ())),
            preferred_element_type=jnp.float32)
        transposed = jax.lax.dot_general(
            m2, xa_i, (((0,), (0,)), ((), ())),
            preferred_element_type=jnp.float32)
        comp = jax.lax.dot_general(
            m2[:, :blk], xa_i, (((1,), (0,)), ((), ())),
            preferred_element_type=jnp.float32)

        s2_ref[lo_r:Kn, :] += transposed
        s2_ref[lo_r:hi_r, :] += direct - comp

    for I in range(n_blocks):
        lo_r, hi_r = I * blk, (I + 1) * blk
        S1 = jax.lax.dot_general(
            a_vmem[lo_r:hi_r, :].astype(jnp.bfloat16), xa_ref[...],
            (((1,), (0,)), ((), ())),
            preferred_element_type=jnp.float32)
        S2 = s2_ref[lo_r:hi_r, :]

        c1 = S1[:, Tn - 1:Tn]
        own = (c1 > 0.0).astype(jnp.float32)
        c2 = S2[:, Tn - 1:Tn] - own
        inv1 = 1.0 / jnp.maximum(c1, 1.0)
        inv2 = 1.0 / jnp.maximum(c2, 1.0)
        own_i2 = own * inv2

        xb = x_ref[lo_r:hi_r, :]
        y = jnp.zeros((blk, Tn - P), dtype=jnp.float32)
        for lag in range(1, P + 1):
            al = coef_ref[0, lag - 1]
            b0l = coef_ref[1, lag - 1]
            b1l = coef_ref[2, lag - 1]
            full = ((al - b1l * own_i2) * xb
                    + (b0l * inv1) * S1
                    + (b1l * inv2) * S2)
            lo, hi = P - lag, Tn - lag
            y = y + full[:, lo:hi]
        y_ref[lo_r:hi_r, :] = y


def kernel(X, A, alpha, beta0, beta1):
    Kn, Tn = X.shape
    P = alpha.shape[0]
    coef = jnp.stack([alpha, beta0, beta1]).astype(jnp.float32)

    blk = min(_BLK, Kn)
    body = functools.partial(_gnar_kernel, n_lags=P, blk=blk)

    return pl.pallas_call(
        body,
        grid=(1,),
        in_specs=[
            pl.BlockSpec((3, P), lambda i: (0, 0)),
            pl.BlockSpec(memory_space=pl.ANY),
            pl.BlockSpec((Kn, Tn), lambda i: (0, 0)),
        ],
        out_specs=pl.BlockSpec((Kn, Tn - P), lambda i: (0, 0)),
        out_shape=jax.ShapeDtypeStruct((Kn, Tn - P), jnp.float32),
        scratch_shapes=[
            pltpu.VMEM((Kn, Kn), jnp.float32),
            pltpu.VMEM((Kn, Kn), jnp.float8_e4m3fn),
            pltpu.VMEM((Kn, Tn), jnp.bfloat16),
            pltpu.VMEM((Kn, Tn), jnp.float32),
            pltpu.SemaphoreType.DMA((Kn // blk,)),
        ],
    )(coef, A, X)


# speedup vs baseline: 1.9728x; 1.0213x over previous
import functools

import jax
import jax.numpy as jnp
from jax.experimental import pallas as pl
from jax.experimental.pallas import tpu as pltpu

_BLK = 256


def _gnar_kernel(coef_ref, a_hbm, x_ref, y_ref, a_vmem, a8_ref, xa_ref,
                 s2_ref, sems, *, n_lags: int, blk: int):
    Kn = a_vmem.shape[0]
    Tn = x_ref.shape[1]
    n_blocks = Kn // blk
    P = n_lags

    half_blocks = n_blocks // 2
    split = half_blocks * blk
    copy_hi = pltpu.make_async_copy(
        a_hbm.at[pl.ds(split, Kn - split), :],
        a_vmem.at[pl.ds(split, Kn - split), :], sems.at[0])
    copy_hi.start()
    copy_lo = None
    if split > 0:
        copy_lo = pltpu.make_async_copy(
            a_hbm.at[pl.ds(0, split), :],
            a_vmem.at[pl.ds(0, split), :], sems.at[1])
        copy_lo.start()

    col = jax.lax.broadcasted_iota(jnp.int32, (Kn, Tn), 1)
    xa_ref[...] = jnp.where(col == Tn - 1, 1.0,
                            x_ref[...]).astype(jnp.bfloat16)
    s2_ref[...] = jnp.zeros((Kn, Tn), dtype=jnp.float32)

    for I in range(n_blocks - 1, -1, -1):
        if I == n_blocks - 1:
            copy_hi.wait()
        if copy_lo is not None and I == half_blocks - 1:
            copy_lo.wait()
        lo_r, hi_r = I * blk, (I + 1) * blk
        a8_ref[lo_r:hi_r, :] = a_vmem[lo_r:hi_r, :].astype(
            jnp.float8_e4m3fn)

        a8_i = a8_ref[lo_r:hi_r, :]
        xa_i = xa_ref[lo_r:hi_r, :]

        reach = jax.lax.dot_general(
            a8_i, a8_ref[lo_r:Kn, :], (((1,), (1,)), ((), ())),
            preferred_element_type=jnp.float32)
        m2 = jnp.maximum(
            jnp.minimum(reach, 1.0) - a_vmem[lo_r:hi_r, lo_r:Kn],
            0.0).astype(jnp.bfloat16)

        xa_strip = xa_ref[lo_r:Kn, :]
        direct = jax.lax.dot_general(
            m2, xa_strip, (((1,), (0,)), ((), ())),
            preferred_element_type=jnp.float32)
        transposed = jax.lax.dot_general(
            m2, xa_i, (((0,), (0,)), ((), ())),
            preferred_element_type=jnp.float32)
        comp = jax.lax.dot_general(
            m2[:, :blk], xa_i, (((1,), (0,)), ((), ())),
            preferred_element_type=jnp.float32)

        s2_ref[lo_r:Kn, :] += transposed
        s2_ref[lo_r:hi_r, :] += direct - comp

    for I in range(n_blocks):
        lo_r, hi_r = I * blk, (I + 1) * blk
        S1 = jax.lax.dot_general(
            a_vmem[lo_r:hi_r, :].astype(jnp.bfloat16), xa_ref[...],
            (((1,), (0,)), ((), ())),
            preferred_element_type=jnp.float32)
        S2 = s2_ref[lo_r:hi_r, :]

        c1 = S1[:, Tn - 1:Tn]
        own = (c1 > 0.0).astype(jnp.float32)
        c2 = S2[:, Tn - 1:Tn] - own
        inv1 = 1.0 / jnp.maximum(c1, 1.0)
        inv2 = 1.0 / jnp.maximum(c2, 1.0)
        own_i2 = own * inv2

        xb = x_ref[lo_r:hi_r, :]
        y = jnp.zeros((blk, Tn - P), dtype=jnp.float32)
        for lag in range(1, P + 1):
            al = coef_ref[0, lag - 1]
            b0l = coef_ref[1, lag - 1]
            b1l = coef_ref[2, lag - 1]
            full = ((al - b1l * own_i2) * xb
                    + (b0l * inv1) * S1
                    + (b1l * inv2) * S2)
            lo, hi = P - lag, Tn - lag
            y = y + full[:, lo:hi]
        y_ref[lo_r:hi_r, :] = y


def kernel(X, A, alpha, beta0, beta1):
    Kn, Tn = X.shape
    P = alpha.shape[0]
    coef = jnp.stack([alpha, beta0, beta1]).astype(jnp.float32)

    blk = min(_BLK, Kn)
    body = functools.partial(_gnar_kernel, n_lags=P, blk=blk)

    return pl.pallas_call(
        body,
        grid=(1,),
        in_specs=[
            pl.BlockSpec((3, P), lambda i: (0, 0)),
            pl.BlockSpec(memory_space=pl.ANY),
            pl.BlockSpec((Kn, Tn), lambda i: (0, 0)),
        ],
        out_specs=pl.BlockSpec((Kn, Tn - P), lambda i: (0, 0)),
        out_shape=jax.ShapeDtypeStruct((Kn, Tn - P), jnp.float32),
        scratch_shapes=[
            pltpu.VMEM((Kn, Kn), jnp.float32),
            pltpu.VMEM((Kn, Kn), jnp.float8_e4m3fn),
            pltpu.VMEM((Kn, Tn), jnp.bfloat16),
            pltpu.VMEM((Kn, Tn), jnp.float32),
            pltpu.SemaphoreType.DMA((2,)),
        ],
    )(coef, A, X)


# speedup vs baseline: 1.9788x; 1.0030x over previous
import functools

import jax
import jax.numpy as jnp
from jax.experimental import pallas as pl
from jax.experimental.pallas import tpu as pltpu

_BLK = 256


def _gnar_kernel(coef_ref, a_hbm, x_ref, y_ref, a_vmem, a8_ref, xa_ref,
                 s2_ref, sems, *, n_lags: int, blk: int):
    Kn = a_vmem.shape[0]
    Tn = x_ref.shape[1]
    n_blocks = Kn // blk
    P = n_lags

    n_chunks = 4 if n_blocks % 4 == 0 and n_blocks >= 4 else 1
    bpc = n_blocks // n_chunks
    rows_pc = bpc * blk
    copies = {}
    for c in range(n_chunks - 1, -1, -1):
        rows = pl.ds(c * rows_pc, rows_pc)
        copies[c] = pltpu.make_async_copy(
            a_hbm.at[rows, :], a_vmem.at[rows, :], sems.at[c])
        copies[c].start()

    col = jax.lax.broadcasted_iota(jnp.int32, (Kn, Tn), 1)
    xa_ref[...] = jnp.where(col == Tn - 1, 1.0,
                            x_ref[...]).astype(jnp.bfloat16)
    s2_ref[...] = jnp.zeros((Kn, Tn), dtype=jnp.float32)

    for I in range(n_blocks - 1, -1, -1):
        if I % bpc == bpc - 1:
            copies[I // bpc].wait()
        lo_r, hi_r = I * blk, (I + 1) * blk
        a8_ref[lo_r:hi_r, :] = a_vmem[lo_r:hi_r, :].astype(
            jnp.float8_e4m3fn)

        a8_i = a8_ref[lo_r:hi_r, :]
        xa_i = xa_ref[lo_r:hi_r, :]

        reach = jax.lax.dot_general(
            a8_i, a8_ref[lo_r:Kn, :], (((1,), (1,)), ((), ())),
            preferred_element_type=jnp.float32)
        m2 = jnp.maximum(
            jnp.minimum(reach.astype(jnp.bfloat16), jnp.bfloat16(1))
            - a8_ref[lo_r:hi_r, lo_r:Kn].astype(jnp.bfloat16),
            jnp.bfloat16(0))

        xa_strip = xa_ref[lo_r:Kn, :]
        direct = jax.lax.dot_general(
            m2, xa_strip, (((1,), (0,)), ((), ())),
            preferred_element_type=jnp.float32)
        transposed = jax.lax.dot_general(
            m2, xa_i, (((0,), (0,)), ((), ())),
            preferred_element_type=jnp.float32)
        comp = jax.lax.dot_general(
            m2[:, :blk], xa_i, (((1,), (0,)), ((), ())),
            preferred_element_type=jnp.float32)

        s2_ref[lo_r:Kn, :] += transposed
        s2_ref[lo_r:hi_r, :] += direct - comp

    for I in range(n_blocks):
        lo_r, hi_r = I * blk, (I + 1) * blk
        S1 = jax.lax.dot_general(
            a_vmem[lo_r:hi_r, :].astype(jnp.bfloat16), xa_ref[...],
            (((1,), (0,)), ((), ())),
            preferred_element_type=jnp.float32)
        S2 = s2_ref[lo_r:hi_r, :]

        c1 = S1[:, Tn - 1:Tn]
        own = (c1 > 0.0).astype(jnp.float32)
        c2 = S2[:, Tn - 1:Tn] - own
        inv1 = 1.0 / jnp.maximum(c1, 1.0)
        inv2 = 1.0 / jnp.maximum(c2, 1.0)
        own_i2 = own * inv2

        xb = x_ref[lo_r:hi_r, :]
        y = jnp.zeros((blk, Tn - P), dtype=jnp.float32)
        for lag in range(1, P + 1):
            al = coef_ref[0, lag - 1]
            b0l = coef_ref[1, lag - 1]
            b1l = coef_ref[2, lag - 1]
            full = ((al - b1l * own_i2) * xb
                    + (b0l * inv1) * S1
                    + (b1l * inv2) * S2)
            lo, hi = P - lag, Tn - lag
            y = y + full[:, lo:hi]
        y_ref[lo_r:hi_r, :] = y


def kernel(X, A, alpha, beta0, beta1):
    Kn, Tn = X.shape
    P = alpha.shape[0]
    coef = jnp.stack([alpha, beta0, beta1]).astype(jnp.float32)

    blk = min(_BLK, Kn)
    body = functools.partial(_gnar_kernel, n_lags=P, blk=blk)

    return pl.pallas_call(
        body,
        grid=(1,),
        in_specs=[
            pl.BlockSpec((3, P), lambda i: (0, 0)),
            pl.BlockSpec(memory_space=pl.ANY),
            pl.BlockSpec((Kn, Tn), lambda i: (0, 0)),
        ],
        out_specs=pl.BlockSpec((Kn, Tn - P), lambda i: (0, 0)),
        out_shape=jax.ShapeDtypeStruct((Kn, Tn - P), jnp.float32),
        scratch_shapes=[
            pltpu.VMEM((Kn, Kn), jnp.float32),
            pltpu.VMEM((Kn, Kn), jnp.float8_e4m3fn),
            pltpu.VMEM((Kn, Tn), jnp.bfloat16),
            pltpu.VMEM((Kn, Tn), jnp.float32),
            pltpu.SemaphoreType.DMA((4,)),
        ],
    )(coef, A, X)


# speedup vs baseline: 2.0346x; 1.0282x over previous
import functools

import jax
import jax.numpy as jnp
from jax.experimental import pallas as pl
from jax.experimental.pallas import tpu as pltpu

_BLK = 256


def _gnar_kernel(coef_ref, a_hbm, x_ref, y_ref, a_vmem, a8_ref, xa_ref,
                 s2_ref, sems, *, n_lags: int, blk: int):
    Kn = a_vmem.shape[0]
    Tn = x_ref.shape[1]
    n_blocks = Kn // blk
    P = n_lags

    n_chunks = 4 if n_blocks % 4 == 0 and n_blocks >= 4 else 1
    bpc = n_blocks // n_chunks
    rows_pc = bpc * blk
    copies = {}
    for c in range(n_chunks - 1, -1, -1):
        rows = pl.ds(c * rows_pc, rows_pc)
        copies[c] = pltpu.make_async_copy(
            a_hbm.at[rows, :], a_vmem.at[rows, :], sems.at[c])
        copies[c].start()

    col = jax.lax.broadcasted_iota(jnp.int32, (Kn, Tn), 1)
    xa_ref[...] = jnp.where(col == Tn - 1, 1.0,
                            x_ref[...]).astype(jnp.bfloat16)
    s2_ref[...] = jnp.zeros((Kn, Tn), dtype=jnp.float32)

    for I in range(n_blocks - 1, -1, -1):
        if I % bpc == bpc - 1:
            copies[I // bpc].wait()
        lo_r, hi_r = I * blk, (I + 1) * blk
        a8_ref[lo_r:hi_r, :] = a_vmem[lo_r:hi_r, :].astype(
            jnp.float8_e4m3fn)

        a8_i = a8_ref[lo_r:hi_r, :]
        xa_i = xa_ref[lo_r:hi_r, :]

        reach = jax.lax.dot_general(
            a8_i, a8_ref[lo_r:Kn, :], (((1,), (1,)), ((), ())),
            preferred_element_type=jnp.float32)
        m2 = jnp.maximum(
            jnp.minimum(reach.astype(jnp.bfloat16), jnp.bfloat16(1))
            - a8_ref[lo_r:hi_r, lo_r:Kn].astype(jnp.bfloat16),
            jnp.bfloat16(0))

        xa_strip = xa_ref[lo_r:Kn, :]
        direct = jax.lax.dot_general(
            m2, xa_strip, (((1,), (0,)), ((), ())),
            preferred_element_type=jnp.float32)
        s2_ref[lo_r:hi_r, :] += direct
        if Kn - hi_r > 0:
            transposed = jax.lax.dot_general(
                m2[:, blk:], xa_i, (((0,), (0,)), ((), ())),
                preferred_element_type=jnp.float32)
            s2_ref[hi_r:Kn, :] += transposed

    for I in range(n_blocks):
        lo_r, hi_r = I * blk, (I + 1) * blk
        S1 = jax.lax.dot_general(
            a_vmem[lo_r:hi_r, :].astype(jnp.bfloat16), xa_ref[...],
            (((1,), (0,)), ((), ())),
            preferred_element_type=jnp.float32)
        S2 = s2_ref[lo_r:hi_r, :]

        c1 = S1[:, Tn - 1:Tn]
        own = (c1 > 0.0).astype(jnp.float32)
        c2 = S2[:, Tn - 1:Tn] - own
        inv1 = 1.0 / jnp.maximum(c1, 1.0)
        inv2 = 1.0 / jnp.maximum(c2, 1.0)
        own_i2 = own * inv2

        xb = x_ref[lo_r:hi_r, :]
        y = jnp.zeros((blk, Tn - P), dtype=jnp.float32)
        for lag in range(1, P + 1):
            al = coef_ref[0, lag - 1]
            b0l = coef_ref[1, lag - 1]
            b1l = coef_ref[2, lag - 1]
            full = ((al - b1l * own_i2) * xb
                    + (b0l * inv1) * S1
                    + (b1l * inv2) * S2)
            lo, hi = P - lag, Tn - lag
            y = y + full[:, lo:hi]
        y_ref[lo_r:hi_r, :] = y


def kernel(X, A, alpha, beta0, beta1):
    Kn, Tn = X.shape
    P = alpha.shape[0]
    coef = jnp.stack([alpha, beta0, beta1]).astype(jnp.float32)

    blk = min(_BLK, Kn)
    body = functools.partial(_gnar_kernel, n_lags=P, blk=blk)

    return pl.pallas_call(
        body,
        grid=(1,),
        in_specs=[
            pl.BlockSpec((3, P), lambda i: (0, 0)),
            pl.BlockSpec(memory_space=pl.ANY),
            pl.BlockSpec((Kn, Tn), lambda i: (0, 0)),
        ],
        out_specs=pl.BlockSpec((Kn, Tn - P), lambda i: (0, 0)),
        out_shape=jax.ShapeDtypeStruct((Kn, Tn - P), jnp.float32),
        scratch_shapes=[
            pltpu.VMEM((Kn, Kn), jnp.float32),
            pltpu.VMEM((Kn, Kn), jnp.float8_e4m3fn),
            pltpu.VMEM((Kn, Tn), jnp.bfloat16),
            pltpu.VMEM((Kn, Tn), jnp.float32),
            pltpu.SemaphoreType.DMA((4,)),
        ],
    )(coef, A, X)


# speedup vs baseline: 2.0389x; 1.0021x over previous
import functools

import jax
import jax.numpy as jnp
from jax.experimental import pallas as pl
from jax.experimental.pallas import tpu as pltpu

_BLK = 256


def _gnar_kernel(coef_ref, a_hbm, x_ref, y_ref, a_vmem, a8_ref,
                 xa_ref, s2_ref, sems, *, n_lags: int, blk: int):
    Kn = a_vmem.shape[0]
    Tn = x_ref.shape[1]
    n_blocks = Kn // blk
    P = n_lags

    n_chunks = 4 if n_blocks % 4 == 0 and n_blocks >= 4 else 1
    bpc = n_blocks // n_chunks
    rows_pc = bpc * blk
    copies = {}
    for c in range(n_chunks - 1, -1, -1):
        rows = pl.ds(c * rows_pc, rows_pc)
        copies[c] = pltpu.make_async_copy(
            a_hbm.at[rows, :], a_vmem.at[rows, :], sems.at[c])
        copies[c].start()

    col = jax.lax.broadcasted_iota(jnp.int32, (Kn, Tn), 1)
    xa_ref[...] = jnp.where(col == Tn - 1, 1.0,
                            x_ref[...]).astype(jnp.bfloat16)

    for I in range(n_blocks - 1, -1, -1):
        if I % bpc == bpc - 1:
            copies[I // bpc].wait()
        lo_r, hi_r = I * blk, (I + 1) * blk
        a8_ref[lo_r:hi_r, :] = a_vmem[lo_r:hi_r, :].astype(
            jnp.float8_e4m3fn)

        a8_i = a8_ref[lo_r:hi_r, :]
        xa_i = xa_ref[lo_r:hi_r, :]

        reach = jax.lax.dot_general(
            a8_i, a8_ref[lo_r:Kn, :], (((1,), (1,)), ((), ())),
            preferred_element_type=jnp.float32)
        m2 = jnp.maximum(
            jnp.minimum(reach.astype(jnp.bfloat16), jnp.bfloat16(1))
            - a8_ref[lo_r:hi_r, lo_r:Kn].astype(jnp.bfloat16),
            jnp.bfloat16(0))

        xa_strip = xa_ref[lo_r:Kn, :]
        direct = jax.lax.dot_general(
            m2, xa_strip, (((1,), (0,)), ((), ())),
            preferred_element_type=jnp.float32)
        s2_ref[lo_r:hi_r, :] = direct
        if Kn - hi_r > 0:
            transposed = jax.lax.dot_general(
                m2[:, blk:], xa_i, (((0,), (0,)), ((), ())),
                preferred_element_type=jnp.float32)
            s2_ref[hi_r:Kn, :] += transposed

    for I in range(n_blocks):
        lo_r, hi_r = I * blk, (I + 1) * blk
        S1 = jax.lax.dot_general(
            a_vmem[lo_r:hi_r, :].astype(jnp.bfloat16), xa_ref[...],
            (((1,), (0,)), ((), ())),
            preferred_element_type=jnp.float32)
        S2 = s2_ref[lo_r:hi_r, :]

        c1 = S1[:, Tn - 1:Tn]
        own = (c1 > 0.0).astype(jnp.float32)
        c2 = S2[:, Tn - 1:Tn] - own
        inv1 = 1.0 / jnp.maximum(c1, 1.0)
        inv2 = 1.0 / jnp.maximum(c2, 1.0)
        own_i2 = own * inv2

        xb = x_ref[lo_r:hi_r, :]
        y = jnp.zeros((blk, Tn - P), dtype=jnp.float32)
        for lag in range(1, P + 1):
            al = coef_ref[0, lag - 1]
            b0l = coef_ref[1, lag - 1]
            b1l = coef_ref[2, lag - 1]
            full = ((al - b1l * own_i2) * xb
                    + (b0l * inv1) * S1
                    + (b1l * inv2) * S2)
            lo, hi = P - lag, Tn - lag
            y = y + full[:, lo:hi]
        y_ref[lo_r:hi_r, :] = y


def kernel(X, A, alpha, beta0, beta1):
    Kn, Tn = X.shape
    P = alpha.shape[0]
    coef = jnp.stack([alpha, beta0, beta1]).astype(jnp.float32)

    blk = min(_BLK, Kn)
    body = functools.partial(_gnar_kernel, n_lags=P, blk=blk)

    return pl.pallas_call(
        body,
        grid=(1,),
        in_specs=[
            pl.BlockSpec((3, P), lambda i: (0, 0)),
            pl.BlockSpec(memory_space=pl.ANY),
            pl.BlockSpec((Kn, Tn), lambda i: (0, 0)),
        ],
        out_specs=pl.BlockSpec((Kn, Tn - P), lambda i: (0, 0)),
        out_shape=jax.ShapeDtypeStruct((Kn, Tn - P), jnp.float32),
        scratch_shapes=[
            pltpu.VMEM((Kn, Kn), jnp.float32),
            pltpu.VMEM((Kn, Kn), jnp.float8_e4m3fn),
            pltpu.VMEM((Kn, Tn), jnp.bfloat16),
            pltpu.VMEM((Kn, Tn), jnp.float32),
            pltpu.SemaphoreType.DMA((4,)),
        ],
    )(coef, A, X)
